# baseline (speedup 1.0000x reference)
import jax
import jax.numpy as jnp
import numpy as np
from jax import lax
from jax.experimental import pallas as pl
from jax.experimental.pallas import tpu as pltpu
from jax.experimental.pallas import tpu_sc as plsc

N_LEVELS = 16
NF = 2
LOG2 = 19
BASE = 16
FINEST = 2048
HASH_SIZE = 2 ** LOG2
MASK = HASH_SIZE - 1
_B_GROWTH = np.exp((np.log(FINEST) - np.log(BASE)) / (N_LEVELS - 1))
RES = [min(int(BASE * (_B_GROWTH ** l)), FINEST) for l in range(N_LEVELS)]


def _i32(u):
    u = int(u) & 0xFFFFFFFF
    return u - (1 << 32) if u >= (1 << 31) else u


P1 = _i32(2654435761)
P2 = _i32(805459861)
P3 = _i32(3674653429)

NC = 2
NS = 16
NW = NC * NS
L = 16

N = 262144
NOUT = N_LEVELS * NF
P_PER_W = N // NW
C = 1024
NGRP = C // L
KC = 8 * C


def _body(pos_hbm, tab_hbm, out_hbm, xv, yv, zv, idx1, rows1, outb, sem):
    wid = lax.axis_index("s") * NC + lax.axis_index("c")
    base = wid * P_PER_W
    lanes = lax.iota(jnp.int32, L)
    lanes32 = lanes * NOUT

    def chunk_body(ci, _):
        cb = base + ci * C
        pltpu.sync_copy(pos_hbm.at[pl.ds(cb, C)], xv)
        pltpu.sync_copy(pos_hbm.at[pl.ds(N + cb, C)], yv)
        pltpu.sync_copy(pos_hbm.at[pl.ds(2 * N + cb, C)], zv)

        for lev in range(N_LEVELS):
            rf = jnp.float32(RES[lev] - 1)
            ri = jnp.int32(RES[lev] - 1)
            off = jnp.int32(lev * HASH_SIZE)

            def idx_grp(g, _, rf=rf, ri=ri, off=off):
                s = g * L
                xi = xv[pl.ds(s, L)]
                yi = yv[pl.ds(s, L)]
                zi = zv[pl.ds(s, L)]
                gx = (xi * rf).astype(jnp.int32)
                gy = (yi * rf).astype(jnp.int32)
                gz = (zi * rf).astype(jnp.int32)
                x0 = jnp.maximum(jnp.minimum(gx, ri), 0)
                x1 = jnp.maximum(jnp.minimum(gx + 1, ri), 0)
                y0 = jnp.maximum(jnp.minimum(gy, ri), 0)
                y1 = jnp.maximum(jnp.minimum(gy + 1, ri), 0)
                z0 = jnp.maximum(jnp.minimum(gz, ri), 0)
                z1 = jnp.maximum(jnp.minimum(gz + 1, ri), 0)
                hx0 = x0 * P1
                hx1 = x1 * P1
                hy0 = y0 * P2
                hy1 = y1 * P2
                hz0 = z0 * P3
                hz1 = z1 * P3
                h00 = hx0 ^ hy0
                h01 = hx0 ^ hy1
                h10 = hx1 ^ hy0
                h11 = hx1 ^ hy1
                hs = (h00, h01, h10, h11)
                c = 0
                for dxy in range(4):
                    for dz in range(2):
                        h = hs[dxy] ^ (hz1 if dz else hz0)
                        e = ((h & MASK) + off) << 1
                        idx1[pl.ds(c * C + s, L)] = e
                        idx1[pl.ds(KC + c * C + s, L)] = e + 1
                        c += 1
                return 0

            lax.fori_loop(0, NGRP, idx_grp, 0)
            pltpu.async_copy(tab_hbm.at[idx1], rows1, sem).wait()

            def tri_grp(g, _, rf=rf, lev=lev):
                s = g * L
                xi = xv[pl.ds(s, L)]
                yi = yv[pl.ds(s, L)]
                zi = zv[pl.ds(s, L)]
                sx = xi * rf
                sy = yi * rf
                sz = zi * rf
                wx = sx - sx.astype(jnp.int32).astype(jnp.float32)
                wy = sy - sy.astype(jnp.int32).astype(jnp.float32)
                wz = sz - sz.astype(jnp.int32).astype(jnp.float32)
                omx = 1.0 - wx
                omy = 1.0 - wy
                omz = 1.0 - wz

                for f in range(NF):
                    fb = f * KC + s

                    def feat(c, fb=fb):
                        return rows1[pl.ds(fb + c * C, L)]

                    f000 = feat(0)
                    f001 = feat(1)
                    f010 = feat(2)
                    f011 = feat(3)
                    f100 = feat(4)
                    f101 = feat(5)
                    f110 = feat(6)
                    f111 = feat(7)
                    c00 = f000 * omz + f001 * wz
                    c01 = f010 * omz + f011 * wz
                    c10 = f100 * omz + f101 * wz
                    c11 = f110 * omz + f111 * wz
                    c0 = c00 * omy + c01 * wy
                    c1 = c10 * omy + c11 * wy
                    res = c0 * omx + c1 * wx
                    ev = (s * NOUT + 2 * lev + f) + lanes32
                    plsc.store_scatter(outb, [ev], res)
                return 0

            lax.fori_loop(0, NGRP, tri_grp, 0)

        pltpu.sync_copy(outb, out_hbm.at[pl.ds(cb * NOUT, C * NOUT)])
        return 0

    lax.fori_loop(0, P_PER_W // C, chunk_body, 0)


@jax.jit
def _encode_sc(pos_flat, tab_flat):
    mesh = plsc.VectorSubcoreMesh(core_axis_name="c", subcore_axis_name="s")
    return pl.kernel(
        _body,
        out_type=jax.ShapeDtypeStruct((N * NOUT,), jnp.float32),
        mesh=mesh,
        compiler_params=pltpu.CompilerParams(needs_layout_passes=False),
        scratch_types=[
            pltpu.VMEM((C,), jnp.float32),
            pltpu.VMEM((C,), jnp.float32),
            pltpu.VMEM((C,), jnp.float32),
            pltpu.VMEM((NF * KC,), jnp.int32),
            pltpu.VMEM((NF * KC,), jnp.float32),
            pltpu.VMEM((C * NOUT,), jnp.float32),
            pltpu.SemaphoreType.DMA,
        ],
    )(pos_flat, tab_flat)


def kernel(positions, tables):
    pos_flat = positions.T.reshape(3 * N)
    tab_flat = tables.reshape(N_LEVELS * HASH_SIZE * NF)
    return _encode_sc(pos_flat, tab_flat).reshape(N, NOUT)
